# baseline (speedup 1.0000x reference)
import functools

import jax
import jax.numpy as jnp
from jax import lax
from jax.experimental import pallas as pl
from jax.experimental.pallas import tpu as pltpu
from jax.experimental.pallas import tpu_sc as plsc

_ALPHABET = 1000
_VP = 1024
_SEQ = 50
_EMB = 1052
_EMBP = 1056
_ET = _EMBP // 8
_ETSTAGE = 5
_ETPAD = 140
_BATCH = 1024
_BT = _BATCH // 128
_SEQP = 56

_NC, _NS = 2, 16
_NW = _NC * _NS

_mesh = plsc.VectorSubcoreMesh(
    core_axis_name="c", subcore_axis_name="s", num_cores=_NC, num_subcores=_NS
)


@functools.partial(
    pl.kernel,
    out_type=jax.ShapeDtypeStruct((_SEQ, _ET, _BT, 8, 128), jnp.float32),
    mesh=_mesh,
    compiler_params=pltpu.CompilerParams(
        use_tc_tiling_on_sc=False, needs_layout_passes=False),
    scratch_types=[
        pltpu.VMEM((_ETSTAGE, 8, _BT, 128), jnp.float32),
        pltpu.VMEM((_SEQ, _BATCH), jnp.int32),
        pltpu.VMEM((_ETSTAGE, 8, _SEQP), jnp.float32),
        pltpu.VMEM((4, _BT, 8, 128), jnp.float32),
        pltpu.SemaphoreType.DMA,
        pltpu.SemaphoreType.DMA,
        pltpu.SemaphoreType.DMA,
        pltpu.SemaphoreType.DMA,
    ],
)
def _emb_kernel(tokt_hbm, xt_hbm, post_hbm, out_hbm,
                tabv, xv, posv, ring, w0, w1, w2, w3):
    wid = lax.axis_index("s") * _NC + lax.axis_index("c")
    wsems = (w0, w1, w2, w3)
    has5 = wid < 4
    et_lo = jnp.where(has5, 5 * wid, 4 * wid + 4)

    pltpu.sync_copy(tokt_hbm.at[pl.ds(et_lo, _ETSTAGE)], tabv)
    pltpu.sync_copy(xt_hbm, xv)
    pltpu.sync_copy(post_hbm.at[pl.ds(et_lo, _ETSTAGE)], posv)

    def write_wait(r):
        pltpu.make_async_copy(ring.at[r], out_hbm.at[0, 0], wsems[r]).wait()

    def unit(s, et_local):
        r = et_local % 4
        obuf = ring.at[r]
        i0 = jnp.full((16,), et_local, jnp.int32)
        s16 = jnp.full((16,), s, jnp.int32)
        psplat = [
            plsc.load_gather(posv, [i0, jnp.full((16,), e, jnp.int32), s16])
            for e in range(8)
        ]

        def btbody(bt, carry):
            for c in range(8):
                v = xv[s, pl.ds(bt * 128 + c * 16, 16)]
                i1 = lax.shift_right_logical(v, 7)
                i3 = lax.bitwise_and(v, 127)
                for e in range(8):
                    i2 = jnp.full((16,), e, jnp.int32)
                    g = plsc.load_gather(tabv, [i0, i1, i2, i3])
                    obuf[bt, e, pl.ds(c * 16, 16)] = g + psplat[e]
            return carry

        lax.fori_loop(0, _BT, btbody, 0)
        pltpu.async_copy(obuf, out_hbm.at[s, et_lo + et_local], wsems[r])

    def sbody(s, carry):
        for et_local in range(4):
            @pl.when(s > 0)
            def _():
                write_wait(et_local)
            unit(s, et_local)

        @pl.when(has5)
        def _():
            write_wait(0)
            unit(s, 4)
        return carry

    lax.fori_loop(0, _SEQ, sbody, 0)
    for r in range(4):
        write_wait(r)


def kernel(X, token_embedding, position_embedding):
    X = X.astype(jnp.int32)
    xt = X.T

    tt = jnp.pad(token_embedding.T,
                 ((0, _ETPAD * 8 - _EMB), (0, _VP - _ALPHABET)))
    tokt = tt.reshape(_ETPAD, 8, _VP // 128, 128).transpose(0, 2, 1, 3)

    post = jnp.pad(position_embedding.T, ((0, _ETPAD * 8 - _EMB), (0, _SEQP - _SEQ)))
    post = post.reshape(_ETPAD, 8, _SEQP)

    out_tiles = _emb_kernel(tokt, xt, post)
    res = out_tiles.transpose(2, 4, 0, 1, 3).reshape(_BATCH, _SEQ, _EMBP)
    return res[:, :, :_EMB]


# speedup vs baseline: 2.0490x; 2.0490x over previous
import functools

import jax
import jax.numpy as jnp
from jax import lax
from jax.experimental import pallas as pl
from jax.experimental.pallas import tpu as pltpu
from jax.experimental.pallas import tpu_sc as plsc

_ALPHABET = 1000
_VP = 1024
_SEQ = 50
_EMB = 1052
_EMBP = 1056
_ET = _EMBP // 8
_ETSTAGE = 5
_ETPAD = 140
_BATCH = 1024
_BT = _BATCH // 128
_SEQP = 56

_NC, _NS = 2, 16
_NW = _NC * _NS

_mesh = plsc.VectorSubcoreMesh(
    core_axis_name="c", subcore_axis_name="s", num_cores=_NC, num_subcores=_NS
)


@functools.partial(
    pl.kernel,
    out_type=jax.ShapeDtypeStruct((_SEQ, _ET, _BT, 8, 128), jnp.float32),
    mesh=_mesh,
    compiler_params=pltpu.CompilerParams(
        use_tc_tiling_on_sc=False, needs_layout_passes=False),
    scratch_types=[
        pltpu.VMEM((_ETSTAGE, 8, _BT, 128), jnp.float32),
        pltpu.VMEM((_SEQ, _BATCH), jnp.int32),
        pltpu.VMEM((_ETSTAGE, 8, _SEQP), jnp.float32),
        pltpu.VMEM((4, _BT, 8, 128), jnp.float32),
        pltpu.SemaphoreType.DMA,
        pltpu.SemaphoreType.DMA,
        pltpu.SemaphoreType.DMA,
        pltpu.SemaphoreType.DMA,
    ],
)
def _emb_kernel(tokt_hbm, xt_hbm, post_hbm, out_hbm,
                tabv, xv, posv, ring, w0, w1, w2, w3):
    wid = lax.axis_index("s") * _NC + lax.axis_index("c")
    wsems = (w0, w1, w2, w3)
    has5 = wid < 4
    et_lo = jnp.where(has5, 5 * wid, 4 * wid + 4)

    pltpu.sync_copy(tokt_hbm.at[pl.ds(et_lo, _ETSTAGE)], tabv)
    pltpu.sync_copy(xt_hbm, xv)
    pltpu.sync_copy(post_hbm.at[pl.ds(et_lo, _ETSTAGE)], posv)

    def write_wait(r):
        pltpu.make_async_copy(ring.at[r], out_hbm.at[0, 0], wsems[r]).wait()

    def unit(s, et_local):
        r = et_local % 4
        obuf = ring.at[r]
        i0 = jnp.full((16,), et_local, jnp.int32)
        s16 = jnp.full((16,), s, jnp.int32)
        psplat = [
            plsc.load_gather(posv, [i0, jnp.full((16,), e, jnp.int32), s16])
            for e in range(8)
        ]

        def btbody(bt, carry):
            for c in range(8):
                v = xv[s, pl.ds(bt * 128 + c * 16, 16)]
                i1 = lax.shift_right_logical(v, 7)
                i3 = lax.bitwise_and(v, 127)
                gs = [
                    plsc.load_gather(
                        tabv, [i0, i1, jnp.full((16,), e, jnp.int32), i3])
                    for e in range(8)
                ]
                for e in range(8):
                    obuf[bt, e, pl.ds(c * 16, 16)] = gs[e] + psplat[e]
            return carry

        lax.fori_loop(0, _BT, btbody, 0)
        pltpu.async_copy(obuf, out_hbm.at[s, et_lo + et_local], wsems[r])

    def sbody(s, carry):
        for et_local in range(4):
            @pl.when(s > 0)
            def _():
                write_wait(et_local)
            unit(s, et_local)

        @pl.when(has5)
        def _():
            write_wait(0)
            unit(s, 4)
        return carry

    lax.fori_loop(0, _SEQ, sbody, 0)
    for r in range(4):
        write_wait(r)


def kernel(X, token_embedding, position_embedding):
    X = X.astype(jnp.int32)
    xt = X.T

    tt = jnp.pad(token_embedding.T,
                 ((0, _ETPAD * 8 - _EMB), (0, _VP - _ALPHABET)))
    tokt = tt.reshape(_ETPAD, 8, _VP // 128, 128).transpose(0, 2, 1, 3)

    post = jnp.pad(position_embedding.T, ((0, _ETPAD * 8 - _EMB), (0, _SEQP - _SEQ)))
    post = post.reshape(_ETPAD, 8, _SEQP)

    out_tiles = _emb_kernel(tokt, xt, post)
    res = out_tiles.transpose(2, 4, 0, 1, 3).reshape(_BATCH, _SEQ, _EMBP)
    return res[:, :, :_EMB]


# speedup vs baseline: 2.5576x; 1.2482x over previous
import functools

import jax
import jax.numpy as jnp
from jax import lax
from jax.experimental import pallas as pl
from jax.experimental.pallas import tpu as pltpu
from jax.experimental.pallas import tpu_sc as plsc

_ALPHABET = 1000
_VP = 1024
_SEQ = 50
_EMB = 1052
_EMBP = 1056
_ET = _EMBP // 8
_ETSTAGE = 6
_ETPAD = 140
_BATCH = 1024
_BT = _BATCH // 128
_SEQP = 56

_NC, _NS = 2, 16
_NW = _NC * _NS
_UNITS = _SEQ * _ET
_UPW = 207

_mesh = plsc.VectorSubcoreMesh(
    core_axis_name="c", subcore_axis_name="s", num_cores=_NC, num_subcores=_NS
)


@functools.partial(
    pl.kernel,
    out_type=jax.ShapeDtypeStruct((_SEQ, _ET, _BT, 8, 128), jnp.float32),
    mesh=_mesh,
    compiler_params=pltpu.CompilerParams(
        use_tc_tiling_on_sc=False, needs_layout_passes=False),
    scratch_types=[
        pltpu.VMEM((_ETSTAGE, 8 * 8 * 128), jnp.float32),
        pltpu.VMEM((_SEQ, _BATCH), jnp.int32),
        pltpu.VMEM((_ETSTAGE, 8, _SEQP), jnp.float32),
        pltpu.VMEM((3, _BT, 8, 128), jnp.float32),
        pltpu.SemaphoreType.DMA,
        pltpu.SemaphoreType.DMA,
        pltpu.SemaphoreType.DMA,
    ],
)
def _emb_kernel(tokt_hbm, xa_hbm, post_hbm, out_hbm,
                tabv, xav, posv, ring, w0, w1, w2):
    wid = lax.axis_index("s") * _NC + lax.axis_index("c")
    wsems = (w0, w1, w2)
    u0 = _UPW * wid
    et_lo = u0 // _SEQ

    pltpu.sync_copy(tokt_hbm.at[pl.ds(et_lo, _ETSTAGE)], tabv)
    pltpu.sync_copy(xa_hbm, xav)
    pltpu.sync_copy(post_hbm.at[pl.ds(et_lo, _ETSTAGE)], posv)

    def write_wait(r):
        pltpu.make_async_copy(ring.at[r], out_hbm.at[0, 0], wsems[r]).wait()

    def unit(u, r):
        et = jnp.minimum(u // _SEQ, _ET - 1)
        s = u - (u // _SEQ) * _SEQ
        et_local = et - et_lo
        obuf = ring.at[r]
        i0 = jnp.full((16,), et_local, jnp.int32)
        s16 = jnp.full((16,), s, jnp.int32)
        psplat = [
            plsc.load_gather(posv, [i0, jnp.full((16,), e, jnp.int32), s16])
            for e in range(8)
        ]

        def btbody(bt, carry):
            for c in range(8):
                av = xav[s, pl.ds(bt * 128 + c * 16, 16)]
                gs = [
                    plsc.load_gather(
                        tabv, [i0, lax.bitwise_or(av, jnp.int32(e * 128))])
                    for e in range(8)
                ]
                for e in range(8):
                    obuf[bt, e, pl.ds(c * 16, 16)] = gs[e] + psplat[e]
            return carry

        lax.fori_loop(0, _BT, btbody, 0)
        pltpu.async_copy(obuf, out_hbm.at[s, et], wsems[r])

    def gbody(g, carry):
        for k in range(3):
            @pl.when(g > 0)
            def _():
                write_wait(k)
            unit(u0 + 3 * g + k, k)
        return carry

    lax.fori_loop(0, _UPW // 3, gbody, 0)
    for r in range(3):
        write_wait(r)


def kernel(X, token_embedding, position_embedding):
    X = X.astype(jnp.int32)
    xa = ((X >> 7) << 10) | (X & 127)
    xat = xa.T

    tt = jnp.pad(token_embedding.T,
                 ((0, _ETPAD * 8 - _EMB), (0, _VP - _ALPHABET)))
    tokt = tt.reshape(_ETPAD, 8, _VP // 128, 128).transpose(0, 2, 1, 3)
    tokt = tokt.reshape(_ETPAD, 8 * 8 * 128)

    post = jnp.pad(position_embedding.T,
                   ((0, _ETPAD * 8 - _EMB), (0, _SEQP - _SEQ)))
    post = post.reshape(_ETPAD, 8, _SEQP)

    out_tiles = _emb_kernel(tokt, xat, post)
    res = out_tiles.transpose(2, 4, 0, 1, 3).reshape(_BATCH, _SEQ, _EMBP)
    return res[:, :, :_EMB]
